# baseline (speedup 1.0000x reference)
import functools

import jax
import jax.numpy as jnp
from jax import lax
from jax.experimental import pallas as pl
from jax.experimental.pallas import tpu as pltpu
from jax.experimental.pallas import tpu_sc as plsc

N_NODES = 10000
N_EDGES = 320000
D = 128
N_CLASSES = 40

NC = 2
NS = 16
NW = NC * NS
LANES = 16

NPAD = 10240
CHUNK = 64
GW = 60
NBUF = 4
K0 = 300
K1 = 16
EPW_PAD = (K0 + K1) * CHUNK * NS // NW
EPAD = NS * (K0 + K1) * CHUNK
PAD_SRC = N_NODES + 8
PAD_DST = NPAD - 1
ROWS_PER_TILE = NPAD // NS


def _sc_mesh():
    return plsc.VectorSubcoreMesh(
        core_axis_name="c", subcore_axis_name="s", num_cores=NC, num_subcores=NS
    )


def _sc_degrees(dst_flat):
    @functools.partial(
        pl.kernel,
        out_type=jax.ShapeDtypeStruct((NW, NPAD), jnp.float32),
        mesh=_sc_mesh(),
        scratch_types=[
            pltpu.VMEM((EPW_PAD,), jnp.int32),
            pltpu.VMEM((NPAD,), jnp.float32),
        ],
        compiler_params=pltpu.CompilerParams(needs_layout_passes=False),
    )
    def k(dst_hbm, out_hbm, idx_v, acc_v):
        wid = lax.axis_index("s") * jnp.int32(NC) + lax.axis_index("c")
        pltpu.sync_copy(dst_hbm.at[pl.ds(wid * jnp.int32(EPW_PAD), EPW_PAD)], idx_v)

        zeros = jnp.zeros((LANES,), jnp.float32)
        lanes32 = jnp.int32(LANES)

        def zbody(i, carry):
            acc_v[pl.ds(i * lanes32, LANES)] = zeros
            return carry

        lax.fori_loop(jnp.int32(0), jnp.int32(NPAD // LANES), zbody, jnp.int32(0))

        ones = jnp.ones((LANES,), jnp.float32)

        def sbody(i, carry):
            idx = idx_v[pl.ds(i * lanes32, LANES)]
            plsc.addupdate_scatter(acc_v, [idx], ones)
            return carry

        lax.fori_loop(jnp.int32(0), jnp.int32(EPW_PAD // LANES), sbody, jnp.int32(0))
        pltpu.sync_copy(acc_v, out_hbm.at[wid])

    return k(dst_flat)


def _sc_layer(x, src_w, dst_w, zero_rows):
    @functools.partial(
        pl.kernel,
        out_type=jax.ShapeDtypeStruct((NC, NPAD, D), jnp.float32),
        mesh=_sc_mesh(),
        scratch_types=[
            pltpu.VMEM((GW, CHUNK), jnp.int32),
            pltpu.VMEM((GW, CHUNK), jnp.int32),
        ] + [pltpu.VMEM((CHUNK, D), jnp.float32) for _ in range(NBUF)]
        + [
            pltpu.VMEM_SHARED((NPAD, D), jnp.float32),
        ] + [pltpu.SemaphoreType.DMA for _ in range(NBUF)],
    )
    def k(x_hbm, src_hbm, dst_hbm, zero_hbm, out_hbm,
          sidx, didx, *rest):
        bufs = rest[:NBUF]
        acc = rest[NBUF]
        sems = rest[NBUF + 1:]
        cid = lax.axis_index("c")
        sid = lax.axis_index("s")
        wid = cid * jnp.int32(NS) + sid
        rbase = sid * jnp.int32(ROWS_PER_TILE)
        pltpu.sync_copy(zero_hbm, rest[0])

        def zfill(r, carry):
            pltpu.sync_copy(
                rest[0],
                acc.at[pl.ds(rbase + r * jnp.int32(CHUNK), CHUNK)])
            return carry

        lax.fori_loop(jnp.int32(0), jnp.int32(ROWS_PER_TILE // CHUNK),
                      zfill, jnp.int32(0))
        plsc.subcore_barrier()

        def gather(j, b):
            pltpu.make_async_copy(x_hbm.at[sidx.at[j]], bufs[b], sems[b]).start()

        def wait(b):
            pltpu.make_async_copy(x_hbm.at[sidx.at[jnp.int32(0)]],
                                  bufs[b], sems[b]).wait()

        def scatter(j, b):
            pltpu.sync_copy(bufs[b], acc.at[didx.at[j]], add=True)

        glimit = lax.select(cid == jnp.int32(0), jnp.int32(GW), jnp.int32(K1))
        ntrip = lax.select(cid == jnp.int32(0),
                           jnp.int32(GW // NBUF), jnp.int32(K1 // NBUF))
        ngroups = lax.select(cid == jnp.int32(0),
                             jnp.int32(K0 // GW), jnp.int32(1))

        def group(g, carry):
            pltpu.sync_copy(src_hbm.at[wid, g], sidx)
            pltpu.sync_copy(dst_hbm.at[wid, g], didx)
            for b in range(NBUF - 1):
                gather(jnp.int32(b), b)

            def body(i, carry2):
                j0 = i * jnp.int32(NBUF)
                for b in range(NBUF):
                    j = j0 + jnp.int32(b)
                    jpre = j + jnp.int32(NBUF - 1)

                    @pl.when(jpre < glimit)
                    def _():
                        gather(jpre, (b + NBUF - 1) % NBUF)

                    wait(b)
                    scatter(j, b)
                return carry2

            lax.fori_loop(jnp.int32(0), ntrip, body, jnp.int32(0))
            return carry

        lax.fori_loop(jnp.int32(0), ngroups, group, jnp.int32(0))
        plsc.subcore_barrier()
        pltpu.sync_copy(acc.at[pl.ds(rbase, ROWS_PER_TILE)],
                        out_hbm.at[cid, pl.ds(rbase, ROWS_PER_TILE)])

    return k(x, src_w, dst_w, zero_rows)


_BLK = 1024


def _tc_norm_scale(parts, h_pad):
    def body(p_ref, h_ref, norm_ref, x0_ref):
        degs = jnp.sum(p_ref[...], axis=0)
        n = lax.rsqrt(jnp.maximum(degs, 1.0))
        norm_ref[...] = n[:, None]
        x0_ref[...] = h_ref[...] * n[:, None]

    return pl.pallas_call(
        body,
        grid=(NPAD // _BLK,),
        in_specs=[
            pl.BlockSpec((NW, _BLK), lambda i: (jnp.int32(0), i)),
            pl.BlockSpec((_BLK, D), lambda i: (i, jnp.int32(0))),
        ],
        out_specs=[
            pl.BlockSpec((_BLK, 1), lambda i: (i, jnp.int32(0))),
            pl.BlockSpec((_BLK, D), lambda i: (i, jnp.int32(0))),
        ],
        out_shape=[
            jax.ShapeDtypeStruct((NPAD, 1), jnp.float32),
            jax.ShapeDtypeStruct((NPAD, D), jnp.float32),
        ],
    )(parts, h_pad)


def _tc_mid(p, norm):
    def body(p_ref, n_ref, o_ref):
        n = n_ref[...]
        o_ref[...] = (p_ref[0] + p_ref[1]) * (n * n)

    return pl.pallas_call(
        body,
        grid=(NPAD // _BLK,),
        in_specs=[
            pl.BlockSpec((NC, _BLK, D), lambda i: (jnp.int32(0), i, jnp.int32(0))),
            pl.BlockSpec((_BLK, 1), lambda i: (i, jnp.int32(0))),
        ],
        out_specs=pl.BlockSpec((_BLK, D), lambda i: (i, jnp.int32(0))),
        out_shape=jax.ShapeDtypeStruct((NPAD, D), jnp.float32),
    )(p, norm)


def _tc_final(p, norm, W1, b1, W2, b2):
    def body(p_ref, n_ref, w1_ref, b1_ref, w2_ref, b2_ref, o_ref):
        h2 = (p_ref[0] + p_ref[1]) * n_ref[...]
        t = jnp.dot(h2, w1_ref[...], preferred_element_type=jnp.float32)
        t = t + b1_ref[...]
        o = jnp.dot(t, w2_ref[...], preferred_element_type=jnp.float32)
        o_ref[...] = o + b2_ref[...]

    blk = 1000
    return pl.pallas_call(
        body,
        grid=(N_NODES // blk,),
        in_specs=[
            pl.BlockSpec((NC, blk, D), lambda i: (jnp.int32(0), i, jnp.int32(0))),
            pl.BlockSpec((blk, 1), lambda i: (i, jnp.int32(0))),
            pl.BlockSpec((D, D), lambda i: (jnp.int32(0), jnp.int32(0))),
            pl.BlockSpec((D,), lambda i: (jnp.int32(0),)),
            pl.BlockSpec((D, N_CLASSES), lambda i: (jnp.int32(0), jnp.int32(0))),
            pl.BlockSpec((N_CLASSES,), lambda i: (jnp.int32(0),)),
        ],
        out_specs=pl.BlockSpec((blk, N_CLASSES), lambda i: (i, jnp.int32(0))),
        out_shape=jax.ShapeDtypeStruct((N_NODES, N_CLASSES), jnp.float32),
    )(p, norm, W1, b1, W2, b2)


def _to_workers(flat, pad_chunks_value):
    a = flat[: NS * K0 * CHUNK].reshape(NS, K0, CHUNK)
    b = flat[NS * K0 * CHUNK:].reshape(NS, K1, CHUNK)
    b = jnp.concatenate(
        [b, jnp.full((NS, K0 - K1, CHUNK), pad_chunks_value, jnp.int32)], axis=1
    )
    return jnp.concatenate([a, b], axis=0).reshape(NW, K0 // GW, GW, CHUNK)


def kernel(h, e, edge_index, W1, b1, W2, b2):
    ei = edge_index.astype(jnp.int32)
    pad2 = jnp.stack([
        jnp.full((EPAD - N_EDGES,), PAD_SRC, jnp.int32),
        jnp.full((EPAD - N_EDGES,), PAD_DST, jnp.int32),
    ])
    ei_pad = jnp.concatenate([ei, pad2], axis=1)
    src_f, dst_f = ei_pad[0], ei_pad[1]
    src_w = _to_workers(src_f, PAD_SRC)
    dst_w = _to_workers(dst_f, PAD_DST)
    zero_rows = jnp.zeros((CHUNK, D), jnp.float32)

    h_pad = jnp.concatenate(
        [h.astype(jnp.float32), jnp.zeros((NPAD - N_NODES, D), jnp.float32)]
    )
    parts = _sc_degrees(dst_f)
    norm, x0 = _tc_norm_scale(parts, h_pad)
    p1 = _sc_layer(x0, src_w, dst_w, zero_rows)
    x1 = _tc_mid(p1, norm)
    p2 = _sc_layer(x1, src_w, dst_w, zero_rows)
    return _tc_final(p2, norm, W1.astype(jnp.float32), b1.astype(jnp.float32),
                     W2.astype(jnp.float32), b2.astype(jnp.float32))


# speedup vs baseline: 1.0619x; 1.0619x over previous
import functools

import jax
import jax.numpy as jnp
from jax import lax
from jax.experimental import pallas as pl
from jax.experimental.pallas import tpu as pltpu
from jax.experimental.pallas import tpu_sc as plsc

N_NODES = 10000
N_EDGES = 320000
D = 128
N_CLASSES = 40

NC = 2
NS = 16
NW = NC * NS
LANES = 16

NPAD = 10240
CHUNK = 64
GW = 60
NBUF = 4
K0 = 300
K1 = 16
EPW_PAD = (K0 + K1) * CHUNK * NS // NW
EPAD = NS * (K0 + K1) * CHUNK
PAD_SRC = N_NODES + 8
PAD_DST = NPAD - 1
ROWS_PER_TILE = NPAD // NS


def _sc_mesh():
    return plsc.VectorSubcoreMesh(
        core_axis_name="c", subcore_axis_name="s", num_cores=NC, num_subcores=NS
    )


def _sc_degrees(dst_flat):
    @functools.partial(
        pl.kernel,
        out_type=jax.ShapeDtypeStruct((NW, NPAD), jnp.float32),
        mesh=_sc_mesh(),
        scratch_types=[
            pltpu.VMEM((EPW_PAD,), jnp.int32),
            pltpu.VMEM((NPAD,), jnp.float32),
        ],
        compiler_params=pltpu.CompilerParams(needs_layout_passes=False),
    )
    def k(dst_hbm, out_hbm, idx_v, acc_v):
        wid = lax.axis_index("s") * jnp.int32(NC) + lax.axis_index("c")
        pltpu.sync_copy(dst_hbm.at[pl.ds(wid * jnp.int32(EPW_PAD), EPW_PAD)], idx_v)

        zeros = jnp.zeros((LANES,), jnp.float32)
        lanes32 = jnp.int32(LANES)

        def zbody(i, carry):
            acc_v[pl.ds(i * lanes32, LANES)] = zeros
            return carry

        lax.fori_loop(jnp.int32(0), jnp.int32(NPAD // LANES), zbody, jnp.int32(0))

        ones = jnp.ones((LANES,), jnp.float32)

        def sbody(i, carry):
            idx = idx_v[pl.ds(i * lanes32, LANES)]
            plsc.addupdate_scatter(acc_v, [idx], ones)
            return carry

        lax.fori_loop(jnp.int32(0), jnp.int32(EPW_PAD // LANES), sbody, jnp.int32(0))
        pltpu.sync_copy(acc_v, out_hbm.at[wid])

    return k(dst_flat)


def _sc_layer(x, src_w, dst_w, zero_rows):
    @functools.partial(
        pl.kernel,
        out_type=jax.ShapeDtypeStruct((NC, NPAD, D), jnp.float32),
        mesh=_sc_mesh(),
        scratch_types=[
            pltpu.VMEM((GW, CHUNK), jnp.int32),
            pltpu.VMEM((GW, CHUNK), jnp.int32),
        ] + [pltpu.VMEM((CHUNK, D), jnp.float32) for _ in range(NBUF)]
        + [
            pltpu.VMEM_SHARED((NPAD, D), jnp.float32),
        ] + [pltpu.SemaphoreType.DMA for _ in range(NBUF)],
    )
    def k(x_hbm, src_hbm, dst_hbm, zero_hbm, out_hbm,
          sidx, didx, *rest):
        bufs = rest[:NBUF]
        acc = rest[NBUF]
        sems = rest[NBUF + 1:]
        cid = lax.axis_index("c")
        sid = lax.axis_index("s")
        wid = cid * jnp.int32(NS) + sid
        rbase = sid * jnp.int32(ROWS_PER_TILE)
        pltpu.sync_copy(zero_hbm, rest[0])

        def zfill(r, carry):
            pltpu.sync_copy(
                rest[0],
                acc.at[pl.ds(rbase + r * jnp.int32(CHUNK), CHUNK)])
            return carry

        lax.fori_loop(jnp.int32(0), jnp.int32(ROWS_PER_TILE // CHUNK),
                      zfill, jnp.int32(0))
        plsc.subcore_barrier()

        def gather(j, b):
            pltpu.make_async_copy(x_hbm.at[sidx.at[j]], bufs[b], sems[b]).start()

        def wait(b):
            pltpu.make_async_copy(x_hbm.at[sidx.at[jnp.int32(0)]],
                                  bufs[b], sems[b]).wait()

        def scatter(j, b):
            pltpu.sync_copy(bufs[b], acc.at[didx.at[j]], add=True)

        glimit = lax.select(cid == jnp.int32(0), jnp.int32(GW), jnp.int32(K1))
        ntrip = lax.select(cid == jnp.int32(0),
                           jnp.int32(GW // NBUF), jnp.int32(K1 // NBUF))
        ngroups = lax.select(cid == jnp.int32(0),
                             jnp.int32(K0 // GW), jnp.int32(1))

        def group(g, carry):
            pltpu.sync_copy(src_hbm.at[wid, g], sidx)
            pltpu.sync_copy(dst_hbm.at[wid, g], didx)
            for b in range(NBUF - 1):
                gather(jnp.int32(b), b)

            def body(i, carry2):
                j0 = i * jnp.int32(NBUF)
                for b in range(NBUF):
                    j = j0 + jnp.int32(b)
                    jpre = j + jnp.int32(NBUF - 1)

                    @pl.when(jpre < glimit)
                    def _():
                        gather(jpre, (b + NBUF - 1) % NBUF)

                    wait(b)
                    scatter(j, b)
                return carry2

            lax.fori_loop(jnp.int32(0), ntrip, body, jnp.int32(0))
            return carry

        lax.fori_loop(jnp.int32(0), ngroups, group, jnp.int32(0))
        plsc.subcore_barrier()
        pltpu.sync_copy(acc.at[pl.ds(rbase, ROWS_PER_TILE)],
                        out_hbm.at[cid, pl.ds(rbase, ROWS_PER_TILE)])

    return k(x, src_w, dst_w, zero_rows)


_BLK = 1024


def _tc_norm_scale(parts, h_pad):
    def body(p_ref, h_ref, norm_ref, x0_ref):
        degs = jnp.sum(p_ref[...], axis=0)
        n = lax.rsqrt(jnp.maximum(degs, 1.0))
        norm_ref[...] = n[:, None]
        x0_ref[...] = h_ref[...] * n[:, None]

    return pl.pallas_call(
        body,
        grid=(NPAD // _BLK,),
        in_specs=[
            pl.BlockSpec((NW, _BLK), lambda i: (jnp.int32(0), i)),
            pl.BlockSpec((_BLK, D), lambda i: (i, jnp.int32(0))),
        ],
        out_specs=[
            pl.BlockSpec((_BLK, 1), lambda i: (i, jnp.int32(0))),
            pl.BlockSpec((_BLK, D), lambda i: (i, jnp.int32(0))),
        ],
        out_shape=[
            jax.ShapeDtypeStruct((NPAD, 1), jnp.float32),
            jax.ShapeDtypeStruct((NPAD, D), jnp.float32),
        ],
    )(parts, h_pad)


def _tc_mid(p, norm):
    def body(p_ref, n_ref, o_ref):
        n = n_ref[...]
        o_ref[...] = (p_ref[0] + p_ref[1]) * (n * n)

    return pl.pallas_call(
        body,
        grid=(NPAD // _BLK,),
        in_specs=[
            pl.BlockSpec((NC, _BLK, D), lambda i: (jnp.int32(0), i, jnp.int32(0))),
            pl.BlockSpec((_BLK, 1), lambda i: (i, jnp.int32(0))),
        ],
        out_specs=pl.BlockSpec((_BLK, D), lambda i: (i, jnp.int32(0))),
        out_shape=jax.ShapeDtypeStruct((NPAD, D), jnp.float32),
    )(p, norm)


def _tc_final(p, norm, W1, b1, W2, b2):
    def body(p_ref, n_ref, w1_ref, b1_ref, w2_ref, b2_ref, o_ref):
        h2 = (p_ref[0] + p_ref[1]) * n_ref[...]
        t = jnp.dot(h2, w1_ref[...], preferred_element_type=jnp.float32)
        t = t + b1_ref[...]
        o = jnp.dot(t, w2_ref[...], preferred_element_type=jnp.float32)
        o_ref[...] = o + b2_ref[...]

    blk = 1000
    return pl.pallas_call(
        body,
        grid=(N_NODES // blk,),
        in_specs=[
            pl.BlockSpec((NC, blk, D), lambda i: (jnp.int32(0), i, jnp.int32(0))),
            pl.BlockSpec((blk, 1), lambda i: (i, jnp.int32(0))),
            pl.BlockSpec((D, D), lambda i: (jnp.int32(0), jnp.int32(0))),
            pl.BlockSpec((D,), lambda i: (jnp.int32(0),)),
            pl.BlockSpec((D, N_CLASSES), lambda i: (jnp.int32(0), jnp.int32(0))),
            pl.BlockSpec((N_CLASSES,), lambda i: (jnp.int32(0),)),
        ],
        out_specs=pl.BlockSpec((blk, N_CLASSES), lambda i: (i, jnp.int32(0))),
        out_shape=jax.ShapeDtypeStruct((N_NODES, N_CLASSES), jnp.float32),
    )(p, norm, W1, b1, W2, b2)


def _to_workers(flat, pad_chunks_value):
    a = flat[: NS * K0 * CHUNK].reshape(NS, K0, CHUNK)
    b = flat[NS * K0 * CHUNK:].reshape(NS, K1, CHUNK)
    b = jnp.concatenate(
        [b, jnp.full((NS, K0 - K1, CHUNK), pad_chunks_value, jnp.int32)], axis=1
    )
    return jnp.concatenate([a, b], axis=0).reshape(NW, K0 // GW, GW, CHUNK)


def kernel(h, e, edge_index, W1, b1, W2, b2):
    src = edge_index[0].astype(jnp.int32)
    dst = edge_index[1].astype(jnp.int32)
    src_f = jnp.concatenate([src, jnp.full((EPAD - N_EDGES,), PAD_SRC, jnp.int32)])
    dst_f = jnp.concatenate([dst, jnp.full((EPAD - N_EDGES,), PAD_DST, jnp.int32)])
    src_w = _to_workers(src_f, PAD_SRC)
    dst_w = _to_workers(dst_f, PAD_DST)
    zero_rows = jnp.zeros((CHUNK, D), jnp.float32)

    h_pad = jnp.concatenate(
        [h.astype(jnp.float32), jnp.zeros((NPAD - N_NODES, D), jnp.float32)]
    )
    parts = _sc_degrees(dst_f)
    norm, x0 = _tc_norm_scale(parts, h_pad)
    p1 = _sc_layer(x0, src_w, dst_w, zero_rows)
    x1 = _tc_mid(p1, norm)
    p2 = _sc_layer(x1, src_w, dst_w, zero_rows)
    return _tc_final(p2, norm, W1.astype(jnp.float32), b1.astype(jnp.float32),
                     W2.astype(jnp.float32), b2.astype(jnp.float32))


# speedup vs baseline: 1.2527x; 1.1797x over previous
import functools

import jax
import jax.numpy as jnp
from jax import lax
from jax.experimental import pallas as pl
from jax.experimental.pallas import tpu as pltpu
from jax.experimental.pallas import tpu_sc as plsc

N_NODES = 10000
N_EDGES = 320000
D = 128
N_CLASSES = 40

NC = 2
NS = 16
NW = NC * NS
LANES = 16

NPAD = 10240
CHUNK = 64
GW = 60
NBUF = 4
K0 = 300
K1 = 13
EPW_PAD = (K0 + K1) * CHUNK * NS // NW
EPAD = NS * (K0 + K1) * CHUNK
PAD_SRC = N_NODES + 8
PAD_DST = NPAD - 1
ROWS_PER_TILE = NPAD // NS


def _sc_mesh():
    return plsc.VectorSubcoreMesh(
        core_axis_name="c", subcore_axis_name="s", num_cores=NC, num_subcores=NS
    )


def _sc_degrees(dst_flat):
    @functools.partial(
        pl.kernel,
        out_type=jax.ShapeDtypeStruct((NW, NPAD), jnp.float32),
        mesh=_sc_mesh(),
        scratch_types=[
            pltpu.VMEM((EPW_PAD,), jnp.int32),
            pltpu.VMEM((NPAD,), jnp.float32),
        ],
        compiler_params=pltpu.CompilerParams(needs_layout_passes=False),
    )
    def k(dst_hbm, out_hbm, idx_v, acc_v):
        wid = lax.axis_index("s") * jnp.int32(NC) + lax.axis_index("c")
        pltpu.sync_copy(dst_hbm.at[pl.ds(wid * jnp.int32(EPW_PAD), EPW_PAD)], idx_v)

        zeros = jnp.zeros((LANES,), jnp.float32)
        lanes32 = jnp.int32(LANES)

        def zbody(i, carry):
            acc_v[pl.ds(i * lanes32, LANES)] = zeros
            return carry

        lax.fori_loop(jnp.int32(0), jnp.int32(NPAD // LANES), zbody, jnp.int32(0))

        ones = jnp.ones((LANES,), jnp.float32)

        def sbody(i, carry):
            idx = idx_v[pl.ds(i * lanes32, LANES)]
            plsc.addupdate_scatter(acc_v, [idx], ones)
            return carry

        lax.fori_loop(jnp.int32(0), jnp.int32(EPW_PAD // LANES), sbody, jnp.int32(0))
        pltpu.sync_copy(acc_v, out_hbm.at[wid])

    return k(dst_flat)


def _sc_layer(x, src_w, dst_w, zero_rows):
    @functools.partial(
        pl.kernel,
        out_type=jax.ShapeDtypeStruct((NC, NPAD, D), jnp.float32),
        mesh=_sc_mesh(),
        scratch_types=[
            pltpu.VMEM((GW, CHUNK), jnp.int32),
            pltpu.VMEM((GW, CHUNK), jnp.int32),
        ] + [pltpu.VMEM((CHUNK, D), jnp.float32) for _ in range(NBUF)]
        + [
            pltpu.VMEM_SHARED((NPAD, D), jnp.float32),
        ] + [pltpu.SemaphoreType.DMA for _ in range(NBUF)],
    )
    def k(x_hbm, src_hbm, dst_hbm, zero_hbm, out_hbm,
          sidx, didx, *rest):
        bufs = rest[:NBUF]
        acc = rest[NBUF]
        sems = rest[NBUF + 1:]
        cid = lax.axis_index("c")
        sid = lax.axis_index("s")
        wid = cid * jnp.int32(NS) + sid
        rbase = sid * jnp.int32(ROWS_PER_TILE)
        pltpu.sync_copy(zero_hbm, rest[0])

        def zfill(r, carry):
            pltpu.sync_copy(
                rest[0],
                acc.at[pl.ds(rbase + r * jnp.int32(CHUNK), CHUNK)])
            return carry

        lax.fori_loop(jnp.int32(0), jnp.int32(ROWS_PER_TILE // CHUNK),
                      zfill, jnp.int32(0))
        plsc.subcore_barrier()

        def gather(j, b):
            pltpu.make_async_copy(x_hbm.at[sidx.at[j]], bufs[b], sems[b]).start()

        def wait(b):
            pltpu.make_async_copy(x_hbm.at[sidx.at[jnp.int32(0)]],
                                  bufs[b], sems[b]).wait()

        def scatter(j, b):
            pltpu.sync_copy(bufs[b], acc.at[didx.at[j]], add=True)

        glimit = lax.select(cid == jnp.int32(0), jnp.int32(GW), jnp.int32(K1))
        ntrip = lax.select(cid == jnp.int32(0),
                           jnp.int32(GW // NBUF),
                           jnp.int32((K1 + NBUF - 1) // NBUF))
        ngroups = lax.select(cid == jnp.int32(0),
                             jnp.int32(K0 // GW), jnp.int32(1))

        def group(g, carry):
            pltpu.sync_copy(src_hbm.at[wid, g], sidx)
            pltpu.sync_copy(dst_hbm.at[wid, g], didx)
            for b in range(NBUF - 1):
                gather(jnp.int32(b), b)

            def body(i, carry2):
                j0 = i * jnp.int32(NBUF)
                for b in range(NBUF):
                    j = j0 + jnp.int32(b)
                    jpre = j + jnp.int32(NBUF - 1)

                    @pl.when(jpre < glimit)
                    def _():
                        gather(jpre, (b + NBUF - 1) % NBUF)

                    @pl.when(j < glimit)
                    def _():
                        wait(b)
                        scatter(j, b)
                return carry2

            lax.fori_loop(jnp.int32(0), ntrip, body, jnp.int32(0))
            return carry

        lax.fori_loop(jnp.int32(0), ngroups, group, jnp.int32(0))
        plsc.subcore_barrier()
        pltpu.sync_copy(acc.at[pl.ds(rbase, ROWS_PER_TILE)],
                        out_hbm.at[cid, pl.ds(rbase, ROWS_PER_TILE)])

    return k(x, src_w, dst_w, zero_rows)


_BLK = 1024


def _tc_norm_scale(parts, h_pad):
    def body(p_ref, h_ref, norm_ref, x0_ref):
        degs = jnp.sum(p_ref[...], axis=0)
        n = lax.rsqrt(jnp.maximum(degs, 1.0))
        norm_ref[...] = n[:, None]
        x0_ref[...] = h_ref[...] * n[:, None]

    return pl.pallas_call(
        body,
        grid=(NPAD // _BLK,),
        in_specs=[
            pl.BlockSpec((NW, _BLK), lambda i: (jnp.int32(0), i)),
            pl.BlockSpec((_BLK, D), lambda i: (i, jnp.int32(0))),
        ],
        out_specs=[
            pl.BlockSpec((_BLK, 1), lambda i: (i, jnp.int32(0))),
            pl.BlockSpec((_BLK, D), lambda i: (i, jnp.int32(0))),
        ],
        out_shape=[
            jax.ShapeDtypeStruct((NPAD, 1), jnp.float32),
            jax.ShapeDtypeStruct((NPAD, D), jnp.float32),
        ],
    )(parts, h_pad)


def _tc_mid(p, norm):
    def body(p_ref, n_ref, o_ref):
        n = n_ref[...]
        o_ref[...] = (p_ref[0] + p_ref[1]) * (n * n)

    return pl.pallas_call(
        body,
        grid=(NPAD // _BLK,),
        in_specs=[
            pl.BlockSpec((NC, _BLK, D), lambda i: (jnp.int32(0), i, jnp.int32(0))),
            pl.BlockSpec((_BLK, 1), lambda i: (i, jnp.int32(0))),
        ],
        out_specs=pl.BlockSpec((_BLK, D), lambda i: (i, jnp.int32(0))),
        out_shape=jax.ShapeDtypeStruct((NPAD, D), jnp.float32),
    )(p, norm)


def _tc_final(p, norm, W1, b1, W2, b2):
    def body(p_ref, n_ref, w1_ref, b1_ref, w2_ref, b2_ref, o_ref):
        h2 = (p_ref[0] + p_ref[1]) * n_ref[...]
        t = jnp.dot(h2, w1_ref[...], preferred_element_type=jnp.float32)
        t = t + b1_ref[...]
        o = jnp.dot(t, w2_ref[...], preferred_element_type=jnp.float32)
        o_ref[...] = o + b2_ref[...]

    blk = 1000
    return pl.pallas_call(
        body,
        grid=(N_NODES // blk,),
        in_specs=[
            pl.BlockSpec((NC, blk, D), lambda i: (jnp.int32(0), i, jnp.int32(0))),
            pl.BlockSpec((blk, 1), lambda i: (i, jnp.int32(0))),
            pl.BlockSpec((D, D), lambda i: (jnp.int32(0), jnp.int32(0))),
            pl.BlockSpec((D,), lambda i: (jnp.int32(0),)),
            pl.BlockSpec((D, N_CLASSES), lambda i: (jnp.int32(0), jnp.int32(0))),
            pl.BlockSpec((N_CLASSES,), lambda i: (jnp.int32(0),)),
        ],
        out_specs=pl.BlockSpec((blk, N_CLASSES), lambda i: (i, jnp.int32(0))),
        out_shape=jax.ShapeDtypeStruct((N_NODES, N_CLASSES), jnp.float32),
    )(p, norm, W1, b1, W2, b2)


def _to_workers(flat, pad_chunks_value):
    a = flat[: NS * K0 * CHUNK].reshape(NS, K0, CHUNK)
    b = flat[NS * K0 * CHUNK:].reshape(NS, K1, CHUNK)
    b = jnp.concatenate(
        [b, jnp.full((NS, K0 - K1, CHUNK), pad_chunks_value, jnp.int32)], axis=1
    )
    return jnp.concatenate([a, b], axis=0).reshape(NW, K0 // GW, GW, CHUNK)


def kernel(h, e, edge_index, W1, b1, W2, b2):
    src = edge_index[0].astype(jnp.int32)
    dst = edge_index[1].astype(jnp.int32)
    src_f = jnp.concatenate([src, jnp.full((EPAD - N_EDGES,), PAD_SRC, jnp.int32)])
    dst_f = jnp.concatenate([dst, jnp.full((EPAD - N_EDGES,), PAD_DST, jnp.int32)])
    src_w = _to_workers(src_f, PAD_SRC)
    dst_w = _to_workers(dst_f, PAD_DST)
    zero_rows = jnp.zeros((CHUNK, D), jnp.float32)

    h_pad = jnp.concatenate(
        [h.astype(jnp.float32), jnp.zeros((NPAD - N_NODES, D), jnp.float32)]
    )
    parts = _sc_degrees(dst_f)
    norm, x0 = _tc_norm_scale(parts, h_pad)
    p1 = _sc_layer(x0, src_w, dst_w, zero_rows)
    x1 = _tc_mid(p1, norm)
    p2 = _sc_layer(x1, src_w, dst_w, zero_rows)
    return _tc_final(p2, norm, W1.astype(jnp.float32), b1.astype(jnp.float32),
                     W2.astype(jnp.float32), b2.astype(jnp.float32))
